# Initial kernel scaffold; baseline (speedup 1.0000x reference)
import jax
import jax.numpy as jnp
from jax.experimental import pallas as pl


def kernel(query, weight):
    raise NotImplementedError("write your pallas kernel here")


# speedup vs baseline: 315.8071x; 315.8071x over previous
import jax
import jax.numpy as jnp
from jax.experimental import pallas as pl
from jax.experimental.pallas import tpu as pltpu

_DIM = 2048
_NCLS = 512
_NQ = 1024
_NBITS = 8
_NLEV = 8
_CHUNK = 256
_NCHUNK = _DIM // _CHUNK
_DTOT = _DIM * _NBITS


def _stats_body(q_ref, w_ref, qn_ref, wn_ref, m_ref, qss, wss, wmn, wmx):
    i = pl.program_id(0)

    @pl.when(i == 0)
    def _init():
        qss[:] = jnp.zeros_like(qss)
        wss[:] = jnp.zeros_like(wss)
        wmn[:] = jnp.full_like(wmn, jnp.inf)
        wmx[:] = jnp.full_like(wmx, -jnp.inf)

    q = q_ref[:]
    w = w_ref[:]
    qss[:] += jnp.sum(q * q, axis=1, keepdims=True)
    wss[:] += jnp.sum(w * w, axis=1, keepdims=True)
    wmn[:] = jnp.minimum(wmn[:], jnp.min(w, axis=1, keepdims=True))
    wmx[:] = jnp.maximum(wmx[:], jnp.max(w, axis=1, keepdims=True))

    @pl.when(i == _NCHUNK - 1)
    def _fin():
        qn_ref[:] = jnp.sqrt(qss[:])
        wnorm = jnp.sqrt(wss[:])
        wn_ref[:] = wnorm
        gmin = jnp.min(wmn[:] / wnorm)
        gmax = jnp.max(wmx[:] / wnorm)
        step = (gmax - gmin) / (_NLEV - 1)
        k = jax.lax.broadcasted_iota(jnp.int32, (1, 8), 1).astype(jnp.float32)
        m_ref[:] = gmin + (k + 0.5) * step


def _encmm_body(q_ref, w_ref, qn_ref, wn_ref, m_ref, out_ref, sq, sw):
    i = pl.program_id(0)

    @pl.when(i == 0)
    def _init():
        out_ref[:] = jnp.zeros_like(out_ref)
        sq[:] = jnp.zeros_like(sq)
        sw[:] = jnp.zeros_like(sw)

    qn = q_ref[:] / qn_ref[:]
    wn = w_ref[:] / wn_ref[:]
    qc = [(qn > m_ref[0, k]).astype(jnp.float32) for k in range(7)]
    wc = [(wn > m_ref[0, k]).astype(jnp.float32) for k in range(7)]
    ones_q = qc[0] + qc[1] + qc[2] + qc[3] + qc[3] + qc[4] + qc[5] + qc[6]
    ones_w = wc[0] + wc[1] + wc[2] + wc[3] + wc[3] + wc[4] + wc[5] + wc[6]
    sq[:] += jnp.sum(ones_q, axis=1, keepdims=True)
    sw[:] += jnp.sum(ones_w, axis=1).reshape(1, _NCLS)
    enc_q = jnp.concatenate(qc, axis=1).astype(jnp.bfloat16)
    wc[3] = wc[3] + wc[3]
    enc_w = jnp.concatenate(wc, axis=1).astype(jnp.bfloat16)
    out_ref[:] += jax.lax.dot_general(
        enc_q, enc_w,
        dimension_numbers=(((1,), (1,)), ((), ())),
        preferred_element_type=jnp.float32,
    )

    @pl.when(i == _NCHUNK - 1)
    def _fin():
        out_ref[:] = (_DTOT + 2.0 * out_ref[:]) - sq[:] - sw[:]


def kernel(query, weight):
    qnorm, wnorm, mids = pl.pallas_call(
        _stats_body,
        grid=(_NCHUNK,),
        in_specs=[
            pl.BlockSpec((_NQ, _CHUNK), lambda i: (0, i)),
            pl.BlockSpec((_NCLS, _CHUNK), lambda i: (0, i)),
        ],
        out_specs=[
            pl.BlockSpec((_NQ, 1), lambda i: (0, 0)),
            pl.BlockSpec((_NCLS, 1), lambda i: (0, 0)),
            pl.BlockSpec((1, 8), lambda i: (0, 0)),
        ],
        out_shape=[
            jax.ShapeDtypeStruct((_NQ, 1), jnp.float32),
            jax.ShapeDtypeStruct((_NCLS, 1), jnp.float32),
            jax.ShapeDtypeStruct((1, 8), jnp.float32),
        ],
        scratch_shapes=[
            pltpu.VMEM((_NQ, 1), jnp.float32),
            pltpu.VMEM((_NCLS, 1), jnp.float32),
            pltpu.VMEM((_NCLS, 1), jnp.float32),
            pltpu.VMEM((_NCLS, 1), jnp.float32),
        ],
    )(query, weight)

    logit = pl.pallas_call(
        _encmm_body,
        grid=(_NCHUNK,),
        in_specs=[
            pl.BlockSpec((_NQ, _CHUNK), lambda i: (0, i)),
            pl.BlockSpec((_NCLS, _CHUNK), lambda i: (0, i)),
            pl.BlockSpec((_NQ, 1), lambda i: (0, 0)),
            pl.BlockSpec((_NCLS, 1), lambda i: (0, 0)),
            pl.BlockSpec((1, 8), lambda i: (0, 0)),
        ],
        out_specs=pl.BlockSpec((_NQ, _NCLS), lambda i: (0, 0)),
        out_shape=jax.ShapeDtypeStruct((_NQ, _NCLS), jnp.float32),
        scratch_shapes=[
            pltpu.VMEM((_NQ, 1), jnp.float32),
            pltpu.VMEM((1, _NCLS), jnp.float32),
        ],
    )(query, weight, qnorm, wnorm, mids)
    return logit


# speedup vs baseline: 333.9280x; 1.0574x over previous
import jax
import jax.numpy as jnp
from jax.experimental import pallas as pl
from jax.experimental.pallas import tpu as pltpu

_DIM = 2048
_NCLS = 512
_NQ = 1024
_NLEV = 8
_CHUNK = 256
_NCHUNK = _DIM // _CHUNK
_DTOT = _DIM * 8
_NCH = 7
_KC = _NCH * _CHUNK
_MA = _NQ + 8
_NA = _NCLS + 128


def _body(q_ref, w_ref, out_ref,
          qsq, wsq, wmn, wmx, tq, tw, enc_q, enc_w, acc):
    i = pl.program_id(0)

    @pl.when(i == 0)
    def _init0():
        qsq[:] = jnp.zeros_like(qsq)
        wsq[:] = jnp.zeros_like(wsq)
        wmn[:] = jnp.full_like(wmn, jnp.inf)
        wmx[:] = jnp.full_like(wmx, -jnp.inf)

    @pl.when(i < _NCHUNK)
    def _phase0():
        q = q_ref[:]
        w = w_ref[:]
        qsq[:] += q * q
        wsq[:] += w * w
        wmn[:] = jnp.minimum(wmn[:], w)
        wmx[:] = jnp.maximum(wmx[:], w)

    @pl.when(i == _NCHUNK - 1)
    def _mk_thresholds():
        qnorm = jnp.sqrt(jnp.sum(qsq[:], axis=1, keepdims=True))
        wnorm = jnp.sqrt(jnp.sum(wsq[:], axis=1, keepdims=True))
        gmin = jnp.min(jnp.min(wmn[:], axis=1, keepdims=True) / wnorm)
        gmax = jnp.max(jnp.max(wmx[:], axis=1, keepdims=True) / wnorm)
        step = (gmax - gmin) / (_NLEV - 1)
        k = jax.lax.broadcasted_iota(jnp.int32, (1, 8), 1).astype(jnp.float32)
        mids = gmin + (k + 0.5) * step
        tq[:] = mids * qnorm
        tw[:] = mids * wnorm

    @pl.when(i == _NCHUNK)
    def _init1():
        acc[:] = jnp.zeros_like(acc)
        row08 = jax.lax.broadcasted_iota(jnp.int32, (8, _KC), 0) == 0
        enc_q[_NQ:_MA, :] = jnp.where(row08, 1.0, 0.0).astype(jnp.bfloat16)
        kblk1 = jax.lax.broadcasted_iota(jnp.int32, (128, _KC), 1) // _CHUNK
        row01 = jax.lax.broadcasted_iota(jnp.int32, (128, _KC), 0) == 0
        enc_w[_NCLS:_NA, :] = jnp.where(
            row01, jnp.where(kblk1 == 3, 2.0, 1.0), 0.0).astype(jnp.bfloat16)

    @pl.when(i >= _NCHUNK)
    def _phase1():
        q = q_ref[:]
        w = w_ref[:]
        for k in range(_NCH):
            cq = (q > tq[:, k:k + 1]).astype(jnp.bfloat16)
            cw = (w > tw[:, k:k + 1]).astype(jnp.bfloat16)
            if k == 3:
                cw = cw + cw
            enc_q[0:_NQ, k * _CHUNK:(k + 1) * _CHUNK] = cq
            enc_w[0:_NCLS, k * _CHUNK:(k + 1) * _CHUNK] = cw
        acc[:] += jax.lax.dot_general(
            enc_q[:], enc_w[:],
            dimension_numbers=(((1,), (1,)), ((), ())),
            preferred_element_type=jnp.float32,
        )

    @pl.when(i == 2 * _NCHUNK - 1)
    def _fin():
        sq = acc[0:_NQ, _NCLS:_NCLS + 1]
        sw = acc[_NQ:_NQ + 1, 0:_NCLS]
        out_ref[:] = (_DTOT + 2.0 * acc[0:_NQ, 0:_NCLS]) - sq - sw


def kernel(query, weight):
    return pl.pallas_call(
        _body,
        grid=(2 * _NCHUNK,),
        in_specs=[
            pl.BlockSpec((_NQ, _CHUNK), lambda i: (0, i % _NCHUNK)),
            pl.BlockSpec((_NCLS, _CHUNK), lambda i: (0, i % _NCHUNK)),
        ],
        out_specs=pl.BlockSpec((_NQ, _NCLS), lambda i: (0, 0)),
        out_shape=jax.ShapeDtypeStruct((_NQ, _NCLS), jnp.float32),
        scratch_shapes=[
            pltpu.VMEM((_NQ, _CHUNK), jnp.float32),
            pltpu.VMEM((_NCLS, _CHUNK), jnp.float32),
            pltpu.VMEM((_NCLS, _CHUNK), jnp.float32),
            pltpu.VMEM((_NCLS, _CHUNK), jnp.float32),
            pltpu.VMEM((_NQ, 8), jnp.float32),
            pltpu.VMEM((_NCLS, 8), jnp.float32),
            pltpu.VMEM((_MA, _KC), jnp.bfloat16),
            pltpu.VMEM((_NA, _KC), jnp.bfloat16),
            pltpu.VMEM((_MA, _NA), jnp.float32),
        ],
    )(query, weight)
